# Initial kernel scaffold; baseline (speedup 1.0000x reference)
import jax
import jax.numpy as jnp
from jax.experimental import pallas as pl


def kernel(x_embed, prompt, prompt_key):
    raise NotImplementedError("write your pallas kernel here")


# speedup vs baseline: 1.2419x; 1.2419x over previous
import jax
import jax.numpy as jnp
from jax import lax
from jax.experimental import pallas as pl
from jax.experimental.pallas import tpu as pltpu

B, S, D = 4, 8192, 768
POOL, LEN, TOPK = 1024, 8, 8

_RB = 64
_OB = 3 * _RB
_NSTEP = (TOPK * LEN + S) // _OB


def _copy_score_body(xa, xb, xc, pk, big, idx_out, rsum, acc):
    t = pl.program_id(0)
    big[:, 0:_RB, :] = xa[...]
    big[:, _RB:2 * _RB, :] = xb[...]
    big[:, 2 * _RB:3 * _RB, :] = xc[...]

    s = jnp.sum(xb[...], axis=1) + jnp.sum(xc[...], axis=1)

    @pl.when(t == 0)
    def _():
        acc[...] = s

    @pl.when(t > 0)
    def _():
        acc[...] = acc[...] + s + jnp.sum(xa[...], axis=1)

    @pl.when(t == _NSTEP - 1)
    def _():
        xm = acc[...] * (1.0 / S)
        xn = xm / jnp.maximum(
            jnp.sqrt(jnp.sum(xm * xm, axis=1, keepdims=True)), 1e-12)
        pkv = pk[...]
        pk_inv = 1.0 / jnp.maximum(
            jnp.sqrt(jnp.sum(pkv * pkv, axis=1)), 1e-12)
        g = lax.dot_general(xn, pkv, (((1,), (1,)), ((), ())),
                            preferred_element_type=jnp.float32)
        sim = g * pk_inv[None, :]

        colid = lax.broadcasted_iota(jnp.int32, (B, POOL), 1)
        col8 = lax.broadcasted_iota(jnp.int32, (B, TOPK), 1)
        idxacc = jnp.zeros((B, TOPK), jnp.int32)
        rs = jnp.float32(0.0)
        work = sim
        for k in range(TOPK):
            m = jnp.max(work, axis=1, keepdims=True)
            amin = jnp.min(jnp.where(work == m, colid, 2 * POOL),
                           axis=1, keepdims=True)
            rs = rs + jnp.sum(m)
            idxacc = jnp.where(col8 == k, amin, idxacc)
            work = jnp.where(colid == amin, -jnp.inf, work)
        idx_out[...] = idxacc
        rsum[...] = jnp.reshape(rs * (1.0 / B), (1, 1))


def _gather_body(idx_ref, prompt_blk, big_any, out_blk):
    del idx_ref, big_any
    out_blk[...] = prompt_blk[...]


def kernel(x_embed, prompt, prompt_key):
    n_out_rows = TOPK * LEN + S

    big, idx, rsum = pl.pallas_call(
        _copy_score_body,
        grid=(_NSTEP,),
        in_specs=[
            pl.BlockSpec((B, _RB, D), lambda t: (0, jnp.maximum(3 * t - 1, 0), 0)),
            pl.BlockSpec((B, _RB, D), lambda t: (0, 3 * t, 0)),
            pl.BlockSpec((B, _RB, D), lambda t: (0, 3 * t + 1, 0)),
            pl.BlockSpec((POOL, D), lambda t: (0, 0)),
        ],
        out_specs=[
            pl.BlockSpec((B, _OB, D), lambda t: (0, t, 0)),
            pl.BlockSpec((B, TOPK), lambda t: (0, 0)),
            pl.BlockSpec((1, 1), lambda t: (0, 0)),
        ],
        out_shape=[
            jax.ShapeDtypeStruct((B, n_out_rows, D), jnp.float32),
            jax.ShapeDtypeStruct((B, TOPK), jnp.int32),
            jax.ShapeDtypeStruct((1, 1), jnp.float32),
        ],
        scratch_shapes=[pltpu.VMEM((B, D), jnp.float32)],
        compiler_params=pltpu.CompilerParams(
            dimension_semantics=("arbitrary",)),
    )(x_embed, x_embed, x_embed, prompt_key)

    idx_flat = idx.reshape(-1)

    out = pl.pallas_call(
        _gather_body,
        grid_spec=pltpu.PrefetchScalarGridSpec(
            num_scalar_prefetch=1,
            grid=(B * TOPK,),
            in_specs=[
                pl.BlockSpec((1, LEN, D), lambda i, idx_ref: (idx_ref[i], 0, 0)),
                pl.BlockSpec(memory_space=pl.ANY),
            ],
            out_specs=pl.BlockSpec(
                (1, LEN, D), lambda i, idx_ref: (i // TOPK, i % TOPK, 0)),
        ),
        out_shape=jax.ShapeDtypeStruct((B, n_out_rows, D), jnp.float32),
        input_output_aliases={2: 0},
    )(idx_flat, prompt[0], big)

    return out, rsum[0, 0]
